# Initial kernel scaffold; baseline (speedup 1.0000x reference)
import jax
import jax.numpy as jnp
from jax.experimental import pallas as pl


def kernel(vol, trf):
    raise NotImplementedError("write your pallas kernel here")


# speedup vs baseline: 1.4308x; 1.4308x over previous
import functools
import jax
import jax.numpy as jnp
from jax import lax
from jax.experimental import pallas as pl
from jax.experimental.pallas import tpu as pltpu
from jax.experimental.pallas import tpu_sc as plsc

_H = 224
_W = 224
_D = 224
_S = 56
_N = _H * _W * _D
_NC = 2
_NS = 16
_NW = _NC * _NS
_ROWS = _S * _H
_RPW = _ROWS // _NW
_G = _W // 16
_FMAX = 223.0
_IMAX = 223

_SCRATCH = [
    pltpu.VMEM((_S * 12,), jnp.float32),
    pltpu.VMEM((_G, 128), jnp.int32),
    pltpu.VMEM((_G, 128), jnp.float32),
    pltpu.VMEM((_W,), jnp.float32),
    pltpu.VMEM((_W,), jnp.float32),
    pltpu.VMEM((_W,), jnp.float32),
    pltpu.VMEM((_W,), jnp.float32),
    pltpu.SemaphoreType.DMA,
    pltpu.SemaphoreType.DMA,
]


def _bf16_round(x):
    u = plsc.bitcast(x, jnp.uint32)
    r = (u + jnp.uint32(0x7FFF) + ((u >> jnp.uint32(16)) & jnp.uint32(1))) \
        & jnp.uint32(0xFFFF0000)
    return plsc.bitcast(r, jnp.float32)


def _v2s_body(vol_hbm, trf_hbm, out_hbm,
              trf_v, idx_v, val_v, dx_v, dy_v, dz_v, row_v, gsem, osem):
    wid = lax.axis_index("sub") * _NC + lax.axis_index("core")
    pltpu.sync_copy(trf_hbm, trf_v)

    def row_body(t, carry):
        r = wid * _RPW + t
        s = r // _H
        i = r % _H

        base = s * 12
        c = [plsc.load_gather(trf_v, [jnp.full((16,), base + k, jnp.int32)])
             for k in range(12)]
        one = jnp.float32(1.0)
        a00 = _bf16_round(c[0] + one)
        a11 = _bf16_round(c[5] + one)
        a22 = _bf16_round(c[10] + one)
        c = [_bf16_round(ck) for ck in c]
        fi = jnp.full((16,), i, jnp.int32).astype(jnp.float32)
        fk = jnp.full((16,), 4 * s, jnp.int32).astype(jnp.float32)
        cx = fi * a00 + fk * c[2] + c[3]
        cy = fi * c[4] + fk * c[6] + c[7]
        cz = fi * c[8] + fk * a22 + c[11]
        ax = c[1]
        ay = a11
        az = c[9]

        lane = lax.broadcasted_iota(jnp.int32, (16,), 0).astype(jnp.float32)
        for g in range(_G):
            jv = lane + jnp.float32(g * 16)
            x = jnp.clip(cx + jv * ax, 0.0, _FMAX)
            y = jnp.clip(cy + jv * ay, 0.0, _FMAX)
            z = jnp.clip(cz + jv * az, 0.0, _FMAX)
            x0 = x.astype(jnp.int32)
            y0 = y.astype(jnp.int32)
            z0 = z.astype(jnp.int32)
            dx_v[pl.ds(g * 16, 16)] = x - x0.astype(jnp.float32)
            dy_v[pl.ds(g * 16, 16)] = y - y0.astype(jnp.float32)
            dz_v[pl.ds(g * 16, 16)] = z - z0.astype(jnp.float32)
            x1 = jnp.minimum(x0 + 1, _IMAX)
            y1 = jnp.minimum(y0 + 1, _IMAX)
            z1 = jnp.minimum(z0 + 1, _IMAX)
            xb0 = x0 * (_W * _D)
            xb1 = x1 * (_W * _D)
            yb0 = y0 * _D
            yb1 = y1 * _D
            b00 = xb0 + yb0
            b01 = xb0 + yb1
            b10 = xb1 + yb0
            b11 = xb1 + yb1
            idx_v[g, pl.ds(0, 16)] = b00 + z0
            idx_v[g, pl.ds(16, 16)] = b00 + z1
            idx_v[g, pl.ds(32, 16)] = b01 + z0
            idx_v[g, pl.ds(48, 16)] = b01 + z1
            idx_v[g, pl.ds(64, 16)] = b10 + z0
            idx_v[g, pl.ds(80, 16)] = b10 + z1
            idx_v[g, pl.ds(96, 16)] = b11 + z0
            idx_v[g, pl.ds(112, 16)] = b11 + z1

        copies = [pltpu.make_async_copy(vol_hbm.at[idx_v.at[g]],
                                        val_v.at[g], gsem)
                  for g in range(_G)]
        for cp in copies:
            cp.start()
        for cp in copies:
            cp.wait()

        for g in range(_G):
            v000 = val_v[g, pl.ds(0, 16)]
            v001 = val_v[g, pl.ds(16, 16)]
            v010 = val_v[g, pl.ds(32, 16)]
            v011 = val_v[g, pl.ds(48, 16)]
            v100 = val_v[g, pl.ds(64, 16)]
            v101 = val_v[g, pl.ds(80, 16)]
            v110 = val_v[g, pl.ds(96, 16)]
            v111 = val_v[g, pl.ds(112, 16)]
            dx = dx_v[pl.ds(g * 16, 16)]
            dy = dy_v[pl.ds(g * 16, 16)]
            dz = dz_v[pl.ds(g * 16, 16)]
            v00 = v000 + dz * (v001 - v000)
            v01 = v010 + dz * (v011 - v010)
            v10 = v100 + dz * (v101 - v100)
            v11 = v110 + dz * (v111 - v110)
            v0 = v00 + dy * (v01 - v00)
            v1 = v10 + dy * (v11 - v10)
            row_v[pl.ds(g * 16, 16)] = v0 + dx * (v1 - v0)

        pltpu.async_copy(row_v, out_hbm.at[s, i], osem).wait()
        return carry

    lax.fori_loop(0, _RPW, row_body, 0)


_v2s_kernel = functools.partial(
    pl.kernel,
    out_type=jax.ShapeDtypeStruct((_S, _H, _W), jnp.float32),
    mesh=plsc.VectorSubcoreMesh(core_axis_name="core", subcore_axis_name="sub",
                                num_cores=_NC, num_subcores=_NS),
    scratch_types=_SCRATCH,
    compiler_params=pltpu.CompilerParams(needs_layout_passes=False),
)(_v2s_body)


@jax.jit
def kernel(vol, trf):
    v = vol.reshape(_N)
    t = trf.reshape(_S * 12)
    out3 = _v2s_kernel(v, t)
    return jnp.transpose(out3, (1, 2, 0))[None, ..., None]
